# baseline (speedup 1.0000x reference)
import jax
import jax.numpy as jnp
from jax import lax
from jax.experimental import pallas as pl
from jax.experimental.pallas import tpu as pltpu
from jax.experimental.pallas import tpu_sc as plsc

N = 10000
R = 8
D = 128
NC, NS = 2, 16
LANES = 16
SUB = 128
SUBS_PER_TILE = 80
QS = 16
EP = NC * NS * SUBS_PER_TILE * SUB
ACC_ROWS = 10240
ZROWS = ACC_ROWS // NS
T = 4096
TPW = T // (NC * NS)


def _bf(x):
    return x.astype(jnp.bfloat16).astype(jnp.float32)


BLKT = 2000


def _transform_body(x_ref, coeff_ref, bases_ref, wself_ref, out_ref):
    r = pl.program_id(0)
    rc = jnp.minimum(r, R - 1)
    sel = lax.broadcasted_iota(jnp.int32, (R, 1), 0) == rc
    crow = jnp.sum(jnp.where(sel, _bf(coeff_ref[...]), 0.0), axis=0)
    wc = jnp.sum(crow[:, None, None] * _bf(bases_ref[...]), axis=0)
    w = jnp.where(r < R, wc, wself_ref[...])
    out_ref[0, :, :] = lax.dot_general(
        x_ref[...].astype(jnp.bfloat16), w.astype(jnp.bfloat16),
        (((1,), (0,)), ((), ())), preferred_element_type=jnp.float32)


def _transform(x, coeff, bases, wself):
    return pl.pallas_call(
        _transform_body,
        grid=(R + 1, N // BLKT),
        in_specs=[
            pl.BlockSpec((BLKT, D), lambda r, j: (j, 0)),
            pl.BlockSpec((R, R), lambda r, j: (0, 0)),
            pl.BlockSpec((R, D, D), lambda r, j: (0, 0, 0)),
            pl.BlockSpec((D, D), lambda r, j: (0, 0)),
        ],
        out_specs=pl.BlockSpec((1, BLKT, D), lambda r, j: (r, j, 0)),
        out_shape=jax.ShapeDtypeStruct((R + 1, N, D), jnp.float32),
    )(x, coeff, bases, wself)


def _agg_body(tab, et2d, col2d, row2d, zeros, part,
              et_v, col_v, row_v, idx_a, idx_b, msg_a, msg_b, acc,
              sem_a, sem_b):
    c = lax.axis_index("c")
    s = lax.axis_index("s")
    pltpu.sync_copy(zeros, acc.at[pl.ds(s * ZROWS, ZROWS)])
    plsc.subcore_barrier()
    sub_base = (c * NS + s) * SUBS_PER_TILE

    def compute_idx(j, idx_buf):
        for l in range(SUB // LANES):
            t16 = et_v[j, pl.ds(l * LANES, LANES)]
            c16 = col_v[j, pl.ds(l * LANES, LANES)]
            idx_buf[pl.ds(l * LANES, LANES)] = t16 * N + c16

    def quarter(q, carry):
        qbase = sub_base + q * QS
        pltpu.sync_copy(et2d.at[pl.ds(qbase, QS)], et_v)
        pltpu.sync_copy(col2d.at[pl.ds(qbase, QS)], col_v)
        pltpu.sync_copy(row2d.at[pl.ds(qbase, QS)], row_v)
        compute_idx(0, idx_a)
        pltpu.async_copy(tab.at[idx_a], msg_a, sem_a)

        def pair(i, carry2):
            j0 = 2 * i
            j1 = j0 + 1
            j2 = jnp.minimum(j0 + 2, QS - 1)
            compute_idx(j1, idx_b)
            pltpu.async_copy(tab.at[idx_b], msg_b, sem_b)
            pltpu.make_async_copy(tab.at[idx_a], msg_a, sem_a).wait()
            pltpu.sync_copy(msg_a, acc.at[row_v.at[j0]], add=True)
            compute_idx(j2, idx_a)
            pltpu.async_copy(tab.at[idx_a], msg_a, sem_a)
            pltpu.make_async_copy(tab.at[idx_b], msg_b, sem_b).wait()
            pltpu.sync_copy(msg_b, acc.at[row_v.at[j1]], add=True)
            return carry2

        lax.fori_loop(0, QS // 2, pair, 0)
        pltpu.make_async_copy(tab.at[idx_a], msg_a, sem_a).wait()
        return carry

    lax.fori_loop(0, SUBS_PER_TILE // QS, quarter, 0)
    plsc.subcore_barrier()

    @pl.when(s < NS - 1)
    def _():
        pltpu.sync_copy(acc.at[pl.ds(s * ZROWS, ZROWS)],
                        part.at[c, pl.ds(s * ZROWS, ZROWS)])

    @pl.when(s == NS - 1)
    def _():
        pltpu.sync_copy(acc.at[pl.ds((NS - 1) * ZROWS, N - (NS - 1) * ZROWS)],
                        part.at[c, pl.ds((NS - 1) * ZROWS, N - (NS - 1) * ZROWS)])


def _sc_aggregate(tab_flat, et2d, col2d, row2d, zeros):
    mesh = plsc.VectorSubcoreMesh(core_axis_name="c", subcore_axis_name="s",
                                  num_cores=NC, num_subcores=NS)
    f = pl.kernel(
        _agg_body,
        out_type=jax.ShapeDtypeStruct((NC, N, D), jnp.float32),
        mesh=mesh,
        scratch_types=[
            pltpu.VMEM((QS, SUB), jnp.int32),
            pltpu.VMEM((QS, SUB), jnp.int32),
            pltpu.VMEM((QS, SUB), jnp.int32),
            pltpu.VMEM((SUB,), jnp.int32),
            pltpu.VMEM((SUB,), jnp.int32),
            pltpu.VMEM((SUB, D), jnp.float32),
            pltpu.VMEM((SUB, D), jnp.float32),
            pltpu.VMEM_SHARED((ACC_ROWS, D), jnp.float32),
            pltpu.SemaphoreType.DMA,
            pltpu.SemaphoreType.DMA,
        ],
    )
    return f(tab_flat, et2d, col2d, row2d, zeros)


BLKC = 2000


def _combine_body(self_ref, p_ref, g_ref, b_ref, out_ref):
    y = self_ref[...] + p_ref[0, :, :] + p_ref[1, :, :]
    y = jnp.maximum(y, 0.0)
    mu = jnp.mean(y, axis=1, keepdims=True)
    d = y - mu
    var = jnp.mean(d * d, axis=1, keepdims=True)
    out_ref[...] = d / jnp.sqrt(var + 1e-5) * g_ref[...] + b_ref[...]


def _combine(selfpart, parts, gamma, beta):
    return pl.pallas_call(
        _combine_body,
        grid=(N // BLKC,),
        in_specs=[
            pl.BlockSpec((BLKC, D), lambda j: (j, 0)),
            pl.BlockSpec((2, BLKC, D), lambda j: (0, j, 0)),
            pl.BlockSpec((1, D), lambda j: (0, 0)),
            pl.BlockSpec((1, D), lambda j: (0, 0)),
        ],
        out_specs=pl.BlockSpec((BLKC, D), lambda j: (j, 0)),
        out_shape=jax.ShapeDtypeStruct((N, D), jnp.float32),
    )(selfpart, parts, gamma.reshape(1, D), beta.reshape(1, D))


def _gather7_body(selftab, p0, p1, rel_emb, hid2d, rid2d, tid2d,
                  o_hs, o_hp0, o_hp1, o_ts, o_tp0, o_tp1, o_r,
                  idh, idr, idt, b_hs, b_hp0, b_hp1, b_ts, b_tp0, b_tp1,
                  b_r, sem):
    c = lax.axis_index("c")
    s = lax.axis_index("s")
    w = c * NS + s
    pltpu.sync_copy(hid2d.at[w], idh)
    pltpu.sync_copy(rid2d.at[w], idr)
    pltpu.sync_copy(tid2d.at[w], idt)
    pltpu.async_copy(selftab.at[idh], b_hs, sem)
    pltpu.async_copy(p0.at[idh], b_hp0, sem)
    pltpu.async_copy(p1.at[idh], b_hp1, sem)
    pltpu.async_copy(selftab.at[idt], b_ts, sem)
    pltpu.async_copy(p0.at[idt], b_tp0, sem)
    pltpu.async_copy(p1.at[idt], b_tp1, sem)
    pltpu.async_copy(rel_emb.at[idr], b_r, sem)
    for buf in (b_hs, b_hp0, b_hp1, b_ts, b_tp0, b_tp1, b_r):
        pltpu.make_async_copy(selftab.at[idh], buf, sem).wait()
    sl = pl.ds(w * TPW, TPW)
    pltpu.sync_copy(b_hs, o_hs.at[sl])
    pltpu.sync_copy(b_hp0, o_hp0.at[sl])
    pltpu.sync_copy(b_hp1, o_hp1.at[sl])
    pltpu.sync_copy(b_ts, o_ts.at[sl])
    pltpu.sync_copy(b_tp0, o_tp0.at[sl])
    pltpu.sync_copy(b_tp1, o_tp1.at[sl])
    pltpu.sync_copy(b_r, o_r.at[sl])


def _gather7(selftab, p0, p1, rel_emb, hid2d, rid2d, tid2d):
    mesh = plsc.VectorSubcoreMesh(core_axis_name="c", subcore_axis_name="s",
                                  num_cores=NC, num_subcores=NS)
    out = jax.ShapeDtypeStruct((T, D), jnp.float32)
    f = pl.kernel(
        _gather7_body,
        out_type=(out,) * 7,
        mesh=mesh,
        scratch_types=[
            pltpu.VMEM((TPW,), jnp.int32),
            pltpu.VMEM((TPW,), jnp.int32),
            pltpu.VMEM((TPW,), jnp.int32),
        ] + [pltpu.VMEM((TPW, D), jnp.float32)] * 7 + [
            pltpu.SemaphoreType.DMA,
        ],
    )
    return f(selftab, p0, p1, rel_emb, hid2d, rid2d, tid2d)


def _predict_body(hs_ref, hp0_ref, hp1_ref, ts_ref, tp0_ref, tp1_ref, r_ref,
                  g_ref, be_ref, w1_ref, b1_ref, w2_ref, b2_ref, out_ref):
    def relu_ln(y):
        y = jnp.maximum(y, 0.0)
        mu = jnp.mean(y, axis=1, keepdims=True)
        d = y - mu
        var = jnp.mean(d * d, axis=1, keepdims=True)
        return d / jnp.sqrt(var + 1e-5) * g_ref[...] + be_ref[...]

    h = relu_ln(hs_ref[...] + hp0_ref[...] + hp1_ref[...])
    t = relu_ln(ts_ref[...] + tp0_ref[...] + tp1_ref[...])

    def dot(a, w):
        return lax.dot_general(a.astype(jnp.bfloat16), w.astype(jnp.bfloat16),
                               (((1,), (0,)), ((), ())),
                               preferred_element_type=jnp.float32)
    a = (dot(h, w1_ref[0:D, :]) + dot(r_ref[...], w1_ref[D:2 * D, :])
         + dot(t, w1_ref[2 * D:3 * D, :]) + b1_ref[...])
    a = jnp.maximum(a, 0.0)
    out_ref[...] = jnp.sum(_bf(a) * _bf(w2_ref[...]), axis=1) + b2_ref[0]


def _predict(h, r, t, gamma, beta, W1, b1, W2, b2):
    return pl.pallas_call(
        _predict_body,
        in_specs=[pl.BlockSpec(memory_space=pltpu.VMEM)] * 12
        + [pl.BlockSpec(memory_space=pltpu.SMEM)],
        out_shape=jax.ShapeDtypeStruct((T,), jnp.float32),
    )(*h, *t, r, gamma.reshape(1, D), beta.reshape(1, D),
      W1, b1.reshape(1, D), W2.reshape(1, D), b2)


def kernel(edge_index, edge_type, head_ids, relation_ids, tail_ids,
           entity_emb, relation_emb,
           bases0, coeff0, Wself0, bases1, coeff1, Wself1,
           gamma0, beta0, gamma1, beta1, W1, b1, W2, b2):
    i32 = jnp.int32
    E = edge_type.shape[0]
    pad = EP - E
    row = edge_index[0].astype(i32)
    col = edge_index[1].astype(i32)
    et = edge_type.astype(i32)
    arange_pad = jnp.arange(pad, dtype=i32)
    et2d = jnp.concatenate([et, jnp.zeros((pad,), i32)]).reshape(EP // SUB, SUB)
    col2d = jnp.concatenate([col, arange_pad % N]).reshape(EP // SUB, SUB)
    row2d = jnp.concatenate([row, N + arange_pad % (ACC_ROWS - N)]).reshape(EP // SUB, SUB)
    zeros = jnp.zeros((ZROWS, D), jnp.float32)

    tab0 = _transform(entity_emb, coeff0, bases0, Wself0)
    part0 = _sc_aggregate(tab0.reshape((R + 1) * N, D), et2d, col2d, row2d, zeros)
    x1 = _combine(tab0[R], part0, gamma0, beta0)

    tab1 = _transform(x1, coeff1, bases1, Wself1)
    part1 = _sc_aggregate(tab1.reshape((R + 1) * N, D), et2d, col2d, row2d, zeros)

    hid2d = head_ids.astype(i32).reshape(NC * NS, TPW)
    rid2d = relation_ids.astype(i32).reshape(NC * NS, TPW)
    tid2d = tail_ids.astype(i32).reshape(NC * NS, TPW)
    hs, hp0, hp1, ts, tp0, tp1, rl = _gather7(
        tab1[R], part1[0], part1[1], relation_emb, hid2d, rid2d, tid2d)
    return _predict((hs, hp0, hp1), rl, (ts, tp0, tp1),
                    gamma1, beta1, W1, b1, W2, b2)


# speedup vs baseline: 1.0682x; 1.0682x over previous
import jax
import jax.numpy as jnp
from jax import lax
from jax.experimental import pallas as pl
from jax.experimental.pallas import tpu as pltpu
from jax.experimental.pallas import tpu_sc as plsc

N = 10000
R = 8
D = 128
NC, NS = 2, 16
LANES = 16
SUB = 128
SUBS_PER_TILE = 80
QS = 40
EP = NC * NS * SUBS_PER_TILE * SUB
ACC_ROWS = 10240
ZROWS = ACC_ROWS // NS
T = 4096
TPW = T // (NC * NS)


def _bf(x):
    return x.astype(jnp.bfloat16).astype(jnp.float32)


def _weights_body(coeff_ref, bases_ref, wself_ref, out_ref):
    bases = _bf(bases_ref[...])
    for r in range(R):
        crow = _bf(coeff_ref[r, :])
        out_ref[r, :, :] = jnp.sum(crow[:, None, None] * bases, axis=0)
    out_ref[R, :, :] = wself_ref[...]


def _make_weights(coeff, bases, wself):
    return pl.pallas_call(
        _weights_body,
        out_shape=jax.ShapeDtypeStruct((R + 1, D, D), jnp.float32),
    )(coeff, bases, wself)


BLKT = 2000


def _transform_body(x_ref, w_ref, out_ref):
    out_ref[0, :, :] = lax.dot_general(
        x_ref[...].astype(jnp.bfloat16), w_ref[0, :, :].astype(jnp.bfloat16),
        (((1,), (0,)), ((), ())), preferred_element_type=jnp.float32)


def _transform(x, w9):
    return pl.pallas_call(
        _transform_body,
        grid=(R + 1, N // BLKT),
        in_specs=[
            pl.BlockSpec((BLKT, D), lambda r, j: (j, 0)),
            pl.BlockSpec((1, D, D), lambda r, j: (r, 0, 0)),
        ],
        out_specs=pl.BlockSpec((1, BLKT, D), lambda r, j: (r, j, 0)),
        out_shape=jax.ShapeDtypeStruct((R + 1, N, D), jnp.float32),
    )(x, w9)


def _agg_body(tab, et2d, col2d, row2d, zeros, part,
              et_v, col_v, row_v, idx_a, idx_b, msg_a, msg_b, acc,
              sem_a, sem_b):
    c = lax.axis_index("c")
    s = lax.axis_index("s")
    pltpu.sync_copy(zeros, acc.at[pl.ds(s * ZROWS, ZROWS)])
    plsc.subcore_barrier()
    sub_base = (c * NS + s) * SUBS_PER_TILE

    def compute_idx(j, idx_buf):
        for l in range(SUB // LANES):
            t16 = et_v[j, pl.ds(l * LANES, LANES)]
            c16 = col_v[j, pl.ds(l * LANES, LANES)]
            idx_buf[pl.ds(l * LANES, LANES)] = t16 * N + c16

    def quarter(q, carry):
        qbase = sub_base + q * QS
        pltpu.sync_copy(et2d.at[pl.ds(qbase, QS)], et_v)
        pltpu.sync_copy(col2d.at[pl.ds(qbase, QS)], col_v)
        pltpu.sync_copy(row2d.at[pl.ds(qbase, QS)], row_v)
        compute_idx(0, idx_a)
        pltpu.async_copy(tab.at[idx_a], msg_a, sem_a)

        def pair(i, carry2):
            j0 = 2 * i
            j1 = j0 + 1
            j2 = jnp.minimum(j0 + 2, QS - 1)
            compute_idx(j1, idx_b)
            pltpu.async_copy(tab.at[idx_b], msg_b, sem_b)
            pltpu.make_async_copy(tab.at[idx_a], msg_a, sem_a).wait()
            pltpu.sync_copy(msg_a, acc.at[row_v.at[j0]], add=True)
            compute_idx(j2, idx_a)
            pltpu.async_copy(tab.at[idx_a], msg_a, sem_a)
            pltpu.make_async_copy(tab.at[idx_b], msg_b, sem_b).wait()
            pltpu.sync_copy(msg_b, acc.at[row_v.at[j1]], add=True)
            return carry2

        lax.fori_loop(0, QS // 2, pair, 0)
        pltpu.make_async_copy(tab.at[idx_a], msg_a, sem_a).wait()
        return carry

    lax.fori_loop(0, SUBS_PER_TILE // QS, quarter, 0)
    plsc.subcore_barrier()

    @pl.when(s < NS - 1)
    def _():
        pltpu.sync_copy(acc.at[pl.ds(s * ZROWS, ZROWS)],
                        part.at[c, pl.ds(s * ZROWS, ZROWS)])

    @pl.when(s == NS - 1)
    def _():
        pltpu.sync_copy(acc.at[pl.ds((NS - 1) * ZROWS, N - (NS - 1) * ZROWS)],
                        part.at[c, pl.ds((NS - 1) * ZROWS, N - (NS - 1) * ZROWS)])


def _sc_aggregate(tab_flat, et2d, col2d, row2d, zeros):
    mesh = plsc.VectorSubcoreMesh(core_axis_name="c", subcore_axis_name="s",
                                  num_cores=NC, num_subcores=NS)
    f = pl.kernel(
        _agg_body,
        out_type=jax.ShapeDtypeStruct((NC, N, D), jnp.float32),
        mesh=mesh,
        scratch_types=[
            pltpu.VMEM((QS, SUB), jnp.int32),
            pltpu.VMEM((QS, SUB), jnp.int32),
            pltpu.VMEM((QS, SUB), jnp.int32),
            pltpu.VMEM((SUB,), jnp.int32),
            pltpu.VMEM((SUB,), jnp.int32),
            pltpu.VMEM((SUB, D), jnp.float32),
            pltpu.VMEM((SUB, D), jnp.float32),
            pltpu.VMEM_SHARED((ACC_ROWS, D), jnp.float32),
            pltpu.SemaphoreType.DMA,
            pltpu.SemaphoreType.DMA,
        ],
    )
    return f(tab_flat, et2d, col2d, row2d, zeros)


BLKC = 2000


def _combine_body(self_ref, p_ref, g_ref, b_ref, out_ref):
    y = self_ref[...] + p_ref[0, :, :] + p_ref[1, :, :]
    y = jnp.maximum(y, 0.0)
    mu = jnp.mean(y, axis=1, keepdims=True)
    d = y - mu
    var = jnp.mean(d * d, axis=1, keepdims=True)
    out_ref[...] = d / jnp.sqrt(var + 1e-5) * g_ref[...] + b_ref[...]


def _combine(selfpart, parts, gamma, beta):
    return pl.pallas_call(
        _combine_body,
        grid=(N // BLKC,),
        in_specs=[
            pl.BlockSpec((BLKC, D), lambda j: (j, 0)),
            pl.BlockSpec((2, BLKC, D), lambda j: (0, j, 0)),
            pl.BlockSpec((1, D), lambda j: (0, 0)),
            pl.BlockSpec((1, D), lambda j: (0, 0)),
        ],
        out_specs=pl.BlockSpec((BLKC, D), lambda j: (j, 0)),
        out_shape=jax.ShapeDtypeStruct((N, D), jnp.float32),
    )(selfpart, parts, gamma.reshape(1, D), beta.reshape(1, D))


def _gather3_body(x2, rel_emb, hid2d, rid2d, tid2d, outh, outr, outt,
                  idh, idr, idt, bufh, bufr, buft, sem):
    c = lax.axis_index("c")
    s = lax.axis_index("s")
    w = c * NS + s
    pltpu.sync_copy(hid2d.at[w], idh)
    pltpu.sync_copy(rid2d.at[w], idr)
    pltpu.sync_copy(tid2d.at[w], idt)
    pltpu.async_copy(x2.at[idh], bufh, sem).wait()
    pltpu.async_copy(rel_emb.at[idr], bufr, sem).wait()
    pltpu.async_copy(x2.at[idt], buft, sem).wait()
    pltpu.sync_copy(bufh, outh.at[pl.ds(w * TPW, TPW)])
    pltpu.sync_copy(bufr, outr.at[pl.ds(w * TPW, TPW)])
    pltpu.sync_copy(buft, outt.at[pl.ds(w * TPW, TPW)])


def _gather3(x2, rel_emb, hid2d, rid2d, tid2d):
    mesh = plsc.VectorSubcoreMesh(core_axis_name="c", subcore_axis_name="s",
                                  num_cores=NC, num_subcores=NS)
    out = jax.ShapeDtypeStruct((T, D), jnp.float32)
    f = pl.kernel(
        _gather3_body,
        out_type=(out, out, out),
        mesh=mesh,
        scratch_types=[
            pltpu.VMEM((TPW,), jnp.int32),
            pltpu.VMEM((TPW,), jnp.int32),
            pltpu.VMEM((TPW,), jnp.int32),
            pltpu.VMEM((TPW, D), jnp.float32),
            pltpu.VMEM((TPW, D), jnp.float32),
            pltpu.VMEM((TPW, D), jnp.float32),
            pltpu.SemaphoreType.DMA,
        ],
    )
    return f(x2, rel_emb, hid2d, rid2d, tid2d)


def _predict_body(h_ref, r_ref, t_ref, w1_ref, b1_ref, w2_ref, b2_ref, out_ref):
    def dot(a, w):
        return lax.dot_general(a.astype(jnp.bfloat16), w.astype(jnp.bfloat16),
                               (((1,), (0,)), ((), ())),
                               preferred_element_type=jnp.float32)
    a = (dot(h_ref[...], w1_ref[0:D, :]) + dot(r_ref[...], w1_ref[D:2 * D, :])
         + dot(t_ref[...], w1_ref[2 * D:3 * D, :]) + b1_ref[...])
    a = jnp.maximum(a, 0.0)
    out_ref[...] = jnp.sum(_bf(a) * _bf(w2_ref[...]), axis=1) + b2_ref[0]


def _predict(h, r, t, W1, b1, W2, b2):
    return pl.pallas_call(
        _predict_body,
        in_specs=[
            pl.BlockSpec(memory_space=pltpu.VMEM),
            pl.BlockSpec(memory_space=pltpu.VMEM),
            pl.BlockSpec(memory_space=pltpu.VMEM),
            pl.BlockSpec(memory_space=pltpu.VMEM),
            pl.BlockSpec(memory_space=pltpu.VMEM),
            pl.BlockSpec(memory_space=pltpu.VMEM),
            pl.BlockSpec(memory_space=pltpu.SMEM),
        ],
        out_shape=jax.ShapeDtypeStruct((T,), jnp.float32),
    )(h, r, t, W1, b1.reshape(1, D), W2.reshape(1, D), b2)


def kernel(edge_index, edge_type, head_ids, relation_ids, tail_ids,
           entity_emb, relation_emb,
           bases0, coeff0, Wself0, bases1, coeff1, Wself1,
           gamma0, beta0, gamma1, beta1, W1, b1, W2, b2):
    i32 = jnp.int32
    E = edge_type.shape[0]
    pad = EP - E
    row = edge_index[0].astype(i32)
    col = edge_index[1].astype(i32)
    et = edge_type.astype(i32)
    arange_pad = jnp.arange(pad, dtype=i32)
    et2d = jnp.concatenate([et, jnp.zeros((pad,), i32)]).reshape(EP // SUB, SUB)
    col2d = jnp.concatenate([col, arange_pad % N]).reshape(EP // SUB, SUB)
    row2d = jnp.concatenate([row, N + arange_pad % (ACC_ROWS - N)]).reshape(EP // SUB, SUB)
    zeros = jnp.zeros((ZROWS, D), jnp.float32)

    w9_0 = _make_weights(coeff0, bases0, Wself0)
    tab0 = _transform(entity_emb, w9_0)
    part0 = _sc_aggregate(tab0.reshape((R + 1) * N, D), et2d, col2d, row2d, zeros)
    x1 = _combine(tab0[R], part0, gamma0, beta0)

    w9_1 = _make_weights(coeff1, bases1, Wself1)
    tab1 = _transform(x1, w9_1)
    part1 = _sc_aggregate(tab1.reshape((R + 1) * N, D), et2d, col2d, row2d, zeros)
    x2 = _combine(tab1[R], part1, gamma1, beta1)

    hid2d = head_ids.astype(i32).reshape(NC * NS, TPW)
    rid2d = relation_ids.astype(i32).reshape(NC * NS, TPW)
    tid2d = tail_ids.astype(i32).reshape(NC * NS, TPW)
    h, rl, t = _gather3(x2, relation_emb, hid2d, rid2d, tid2d)
    return _predict(h, rl, t, W1, b1, W2, b2)
